# baseline (speedup 1.0000x reference)
import functools

import jax
import jax.numpy as jnp
from jax import lax
from jax.experimental import pallas as pl
from jax.experimental.pallas import tpu as pltpu
from jax.experimental.pallas import tpu_sc as plsc

N = 50000
NLEV = 4
DIMS = (16, 32, 64, 128)
DTOT = 240
NC, NS = 2, 16
NW = NC * NS
BLK = 128
NBLK = -(-N // BLK)
LAST_BASE = N - BLK
NA, NB1, NB2 = 16, 11, 5
ITERS_A = -(-NBLK // NA)
ITERS_B1 = -(-NBLK // NB1)
ITERS_B2 = -(-NBLK // NB2)


def _body(clvt, w0t, w1t, w2t, w3, outt, i0, i1, i2, i3, w3rows, outb,
          st0, st1, st2, sem):
    wid = lax.axis_index("s") * NC + lax.axis_index("c")
    idx_v = (i0, i1, i2, i3)

    def block_base(blk):
        return lax.min(blk * BLK, LAST_BASE)

    def load_idx(l, base):
        pltpu.sync_copy(clvt.at[l, pl.ds(base, BLK)], idx_v[l])
        for j in range(BLK // 16):
            idx_v[l][pl.ds(j * 16, 16)] = idx_v[l][pl.ds(j * 16, 16)] - 1

    def a_path():
        aid = wid

        def blk_body(t, carry):
            blk = t * NA + aid

            @pl.when(blk < NBLK)
            def _():
                base = block_base(blk)
                load_idx(3, base)
                pltpu.async_copy(w3.at[i3], w3rows, sem).wait()

                def ft_body(ft, carry2):
                    for r in range(16):
                        f = ft * 16 + r
                        fvec = jnp.full((16,), f, jnp.int32)
                        for cg in range(8):
                            cvec = lax.iota(jnp.int32, 16) + (cg * 16)
                            v = plsc.load_gather(w3rows, [cvec, fvec])
                            outb[f, pl.ds(cg * 16, 16)] = v
                    return carry2

                lax.fori_loop(0, 8, ft_body, 0)
                pltpu.sync_copy(
                    outb, outt.at[pl.ds(112, 128), pl.ds(base, BLK)])
            return carry

        lax.fori_loop(0, ITERS_A, blk_body, 0)

    def b_path(bid, n_g, iters, levels, out_off, out_rows):
        pltpu.sync_copy(w0t, st0)
        pltpu.sync_copy(w1t, st1)
        w2_half = 0 if out_off == 0 else 32
        pltpu.sync_copy(w2t.at[pl.ds(w2_half, 32), :], st2)

        def blk_body(t, carry):
            blk = t * n_g + bid

            @pl.when(blk < NBLK)
            def _():
                base = block_base(blk)
                for l, _, _, _ in levels:
                    load_idx(l, base)

                def cg_body(cg, carry2):
                    row = 0
                    for l, stage, nf, foff in levels:
                        ivec = idx_v[l][pl.ds(cg * 16, 16)]
                        for f in range(nf):
                            fvec = jnp.full((16,), f + foff, jnp.int32)
                            v = plsc.load_gather(stage, [fvec, ivec])
                            outb[row + f, pl.ds(cg * 16, 16)] = v
                        row += nf
                    return carry2

                lax.fori_loop(0, 8, cg_body, 0)
                pltpu.sync_copy(
                    outb.at[pl.ds(0, out_rows), :],
                    outt.at[pl.ds(out_off, out_rows), pl.ds(base, BLK)])
            return carry

        lax.fori_loop(0, iters, blk_body, 0)

    pl.when(wid < NA)(a_path)
    pl.when((wid >= NA) & (wid < NA + NB1))(
        lambda: b_path(wid - NA, NB1, ITERS_B1,
                       ((0, st0, 16, 0), (1, st1, 32, 0), (2, st2, 32, 0)),
                       0, 80))
    pl.when(wid >= NA + NB1)(
        lambda: b_path(wid - NA - NB1, NB2, ITERS_B2,
                       ((2, st2, 32, 0),),
                       80, 32))


@jax.jit
def kernel(code_levels, W0, W1, W2, W3):
    mesh = plsc.VectorSubcoreMesh(core_axis_name="c", subcore_axis_name="s")
    f = pl.kernel(
        _body,
        out_type=jax.ShapeDtypeStruct((DTOT, N), jnp.float32),
        mesh=mesh,
        scratch_types=[
            pltpu.VMEM((BLK,), jnp.int32),
            pltpu.VMEM((BLK,), jnp.int32),
            pltpu.VMEM((BLK,), jnp.int32),
            pltpu.VMEM((BLK,), jnp.int32),
            pltpu.VMEM((BLK, DIMS[3]), jnp.float32),
            pltpu.VMEM((BLK, BLK), jnp.float32),
            pltpu.VMEM((DIMS[0], 20), jnp.float32),
            pltpu.VMEM((DIMS[1], 200), jnp.float32),
            pltpu.VMEM((32, 2000), jnp.float32),
            pltpu.SemaphoreType.DMA,
        ],
        compiler_params=pltpu.CompilerParams(
            use_tc_tiling_on_sc=False, needs_layout_passes=False),
    )
    outt = f(code_levels.T, W0.T, W1.T, W2.T, W3)
    return outt.T


# speedup vs baseline: 1.0901x; 1.0901x over previous
import functools

import jax
import jax.numpy as jnp
from jax import lax
from jax.experimental import pallas as pl
from jax.experimental.pallas import tpu as pltpu
from jax.experimental.pallas import tpu_sc as plsc

N = 50000
NLEV = 4
DIMS = (16, 32, 64, 128)
DTOT = 240
NC, NS = 2, 16
NW = NC * NS
BLK = 128
NBLK = -(-N // BLK)
LAST_BASE = N - BLK
NA, NB1, NB2 = 16, 11, 5
ITERS_A = -(-NBLK // NA)
ITERS_B1 = -(-NBLK // NB1)
ITERS_B2 = -(-NBLK // NB2)


def _body(clvt, w0t, w1t, w2t, w3, outt, i0, i1, i2, i3, w3r0, w3r1, outb,
          st0, st1, st2, sem, wsem):
    wid = lax.axis_index("s") * NC + lax.axis_index("c")
    idx_v = (i0, i1, i2, i3)
    iota = lax.iota(jnp.int32, 16)
    col128 = iota * 128

    def block_base(blk):
        return lax.min(blk * BLK, LAST_BASE)

    def dec_idx(l):
        for j in range(BLK // 16):
            idx_v[l][pl.ds(j * 16, 16)] = idx_v[l][pl.ds(j * 16, 16)] - 1

    def a_path():
        aid = wid

        def fire(blk, buf):
            base = block_base(blk)
            pltpu.sync_copy(clvt.at[pl.ds(3 * N + base, BLK)], i3)
            dec_idx(3)
            pltpu.async_copy(w3.at[i3], buf, sem)

        def work(t, buf, other):
            blk = lax.min(t * NA + aid, NBLK - 1)
            pltpu.make_async_copy(w3.at[i3], buf, sem).wait()

            @pl.when(t + 1 < ITERS_A)
            def _():
                fire(lax.min((t + 1) * NA + aid, NBLK - 1), other)

            def cg_body(cg, carry2):
                cvec = iota + cg * 16
                for f in range(128):
                    fvec = jnp.full((16,), f, jnp.int32)
                    v = plsc.load_gather(buf, [cvec, fvec])
                    outb[f, pl.ds(cg * 16, 16)] = v
                return carry2

            lax.fori_loop(0, 8, cg_body, 0)
            pltpu.sync_copy(
                outb, outt.at[pl.ds(112, 128), pl.ds(block_base(blk), BLK)])

        fire(aid, w3r0)

        def blk_body(t, carry):
            pl.when(t % 2 == 0)(lambda: work(t, w3r0, w3r1))
            pl.when(t % 2 == 1)(lambda: work(t, w3r1, w3r0))
            return carry

        lax.fori_loop(0, ITERS_A, blk_body, 0)

    def b_path(bid, n_g, iters, levels, out_off, out_rows, w2_half):
        pltpu.sync_copy(w0t, st0)
        pltpu.sync_copy(w1t, st1)
        pltpu.sync_copy(w2t.at[pl.ds(w2_half * 64000, 64000)], st2)

        def blk_body(t, carry):
            blk = t * n_g + bid

            @pl.when(blk < NBLK)
            def _():
                base = block_base(blk)
                for l, _, _, _ in levels:
                    pltpu.async_copy(
                        clvt.at[pl.ds(l * N + base, BLK)], idx_v[l], wsem)
                for l, _, _, _ in levels:
                    pltpu.make_async_copy(
                        clvt.at[pl.ds(l * N + base, BLK)], idx_v[l],
                        wsem).wait()
                for l, _, _, _ in levels:
                    dec_idx(l)

                def cg_body(cg, carry2):
                    row = 0
                    for l, stage, nf, nv in levels:
                        ivec = idx_v[l][pl.ds(cg * 16, 16)]
                        for f in range(nf):
                            v = plsc.load_gather(stage, [ivec + (f * nv)])
                            outb[row + f, pl.ds(cg * 16, 16)] = v
                        row += nf
                    return carry2

                lax.fori_loop(0, 8, cg_body, 0)
                pltpu.sync_copy(
                    outb.at[pl.ds(0, out_rows), :],
                    outt.at[pl.ds(out_off, out_rows), pl.ds(base, BLK)])
            return carry

        lax.fori_loop(0, iters, blk_body, 0)

    pl.when(wid < NA)(a_path)
    pl.when((wid >= NA) & (wid < NA + NB1))(
        lambda: b_path(wid - NA, NB1, ITERS_B1,
                       ((0, st0, 16, 20), (1, st1, 32, 200),
                        (2, st2, 32, 2000)),
                       0, 80, 0))
    pl.when(wid >= NA + NB1)(
        lambda: b_path(wid - NA - NB1, NB2, ITERS_B2,
                       ((2, st2, 32, 2000),),
                       80, 32, 1))


@jax.jit
def kernel(code_levels, W0, W1, W2, W3):
    mesh = plsc.VectorSubcoreMesh(core_axis_name="c", subcore_axis_name="s")
    f = pl.kernel(
        _body,
        out_type=jax.ShapeDtypeStruct((DTOT, N), jnp.float32),
        mesh=mesh,
        scratch_types=[
            pltpu.VMEM((BLK,), jnp.int32),
            pltpu.VMEM((BLK,), jnp.int32),
            pltpu.VMEM((BLK,), jnp.int32),
            pltpu.VMEM((BLK,), jnp.int32),
            pltpu.VMEM((BLK, DIMS[3]), jnp.float32),
            pltpu.VMEM((BLK, DIMS[3]), jnp.float32),
            pltpu.VMEM((BLK, BLK), jnp.float32),
            pltpu.VMEM((DIMS[0] * 20,), jnp.float32),
            pltpu.VMEM((DIMS[1] * 200,), jnp.float32),
            pltpu.VMEM((32 * 2000,), jnp.float32),
            pltpu.SemaphoreType.DMA,
            pltpu.SemaphoreType.DMA,
        ],
        compiler_params=pltpu.CompilerParams(
            use_tc_tiling_on_sc=False, needs_layout_passes=False),
    )
    outt = f(code_levels.T.reshape(-1), W0.T.reshape(-1), W1.T.reshape(-1),
             W2.T.reshape(-1), W3)
    return outt.T


# speedup vs baseline: 1.1660x; 1.0696x over previous
import functools

import jax
import jax.numpy as jnp
from jax import lax
from jax.experimental import pallas as pl
from jax.experimental.pallas import tpu as pltpu
from jax.experimental.pallas import tpu_sc as plsc

N = 50000
NLEV = 4
DIMS = (16, 32, 64, 128)
DTOT = 240
NC, NS = 2, 16
NW = NC * NS
NA, NB1, NB2 = 16, 11, 5

ABLK = 64
NBLK_A = -(-N // ABLK)
LAST_A = N - ABLK
SLOTS_A = 2 * (-(-NBLK_A // (2 * NA)))

BBLK = 128
NBLK_B = -(-N // BBLK)
LAST_B = N - BBLK
SLOTS_B1 = 2 * (-(-NBLK_B // (2 * NB1)))
SLOTS_B2 = 2 * (-(-NBLK_B // (2 * NB2)))


def _body(clvt, w0t, w1t, w2t, w3, outt,
          ia0, ia1, ib0, ib1, w3ra, w3rb, oa0, oa1, ob0, ob1,
          st0, st1, st2, isem, gsem, wsem):
    wid = lax.axis_index("s") * NC + lax.axis_index("c")
    iota = lax.iota(jnp.int32, 16)

    def dec(ref, n):
        for j in range(n // 16):
            ref[pl.ds(j * 16, 16)] = ref[pl.ds(j * 16, 16)] - 1


    def a_path():
        aid = wid

        def base_a(t):
            return lax.min(lax.min(t, SLOTS_A - 1) * NA + aid,
                           NBLK_A - 1) * ABLK

        def base_a_clamped(t):
            return lax.min(base_a(t), LAST_A)

        def idx_src(t):
            return clvt.at[pl.ds(3 * N + base_a_clamped(t), ABLK)]

        def fire_idx(t, iref):
            pltpu.async_copy(idx_src(t), iref, isem)

        def wait_idx(t, iref):
            pltpu.make_async_copy(idx_src(t), iref, isem).wait()

        def fire_gather(iref, wref):
            pltpu.async_copy(w3.at[iref], wref, gsem)

        def wait_gather(iref, wref):
            pltpu.make_async_copy(w3.at[iref], wref, gsem).wait()

        def out_dst(t):
            return outt.at[pl.ds(112, 128), pl.ds(base_a_clamped(t), ABLK)]

        def transpose(wref, oref):
            def cg_body(cg, carry):
                cvec = iota + cg * 16
                for f in range(128):
                    fvec = jnp.full((16,), f, jnp.int32)
                    v = plsc.load_gather(wref, [cvec, fvec])
                    oref[f, pl.ds(cg * 16, 16)] = v
                return carry

            lax.fori_loop(0, ABLK // 16, cg_body, 0)

        def half(k, t, icur, inxt, wcur, wnxt, ocur):
            wait_idx(t + 1, inxt)
            dec(inxt, ABLK)
            wait_gather(icur, wcur)
            fire_gather(inxt, wnxt)
            fire_idx(t + 2, icur)
            pl.when(k > 0)(
                lambda: pltpu.make_async_copy(ocur, out_dst(t - 2),
                                              wsem).wait())
            transpose(wcur, ocur)
            pltpu.async_copy(ocur, out_dst(t), wsem)

        pltpu.sync_copy(idx_src(0), ia0)
        dec(ia0, ABLK)
        fire_gather(ia0, w3ra)
        fire_idx(1, ia1)

        def blk_body(k, carry):
            half(k, 2 * k, ia0, ia1, w3ra, w3rb, oa0)
            half(k, 2 * k + 1, ia1, ia0, w3rb, w3ra, oa1)
            return carry

        lax.fori_loop(0, SLOTS_A // 2, blk_body, 0)
        pltpu.make_async_copy(idx_src(0), ia0, isem).wait()
        pltpu.make_async_copy(w3.at[ia0], w3ra, gsem).wait()
        pltpu.make_async_copy(oa0, out_dst(SLOTS_A - 2), wsem).wait()
        pltpu.make_async_copy(oa1, out_dst(SLOTS_A - 1), wsem).wait()


    def b_path(bid, n_g, slots, levels, out_off, out_rows, w2_half):
        pltpu.sync_copy(w0t, st0)
        pltpu.sync_copy(w1t, st1)
        pltpu.sync_copy(w2t.at[pl.ds(w2_half * 64000, 64000)], st2)
        nlv = len(levels)

        def base_b(t):
            return lax.min(
                lax.min(lax.min(t, slots - 1) * n_g + bid, NBLK_B - 1)
                * BBLK, LAST_B)

        def idx_src(t, l):
            return clvt.at[pl.ds(l * N + base_b(t), BBLK)]

        def fire_idxset(t, irefs):
            for (l, _, _, _), iref in zip(levels, irefs):
                pltpu.async_copy(idx_src(t, l), iref, isem)

        def wait_idxset(t, irefs):
            for (l, _, _, _), iref in zip(levels, irefs):
                pltpu.make_async_copy(idx_src(t, l), iref, isem).wait()
            for iref in irefs:
                dec(iref, BBLK)

        def out_dst(t):
            return outt.at[pl.ds(out_off, out_rows),
                           pl.ds(base_b(t), BBLK)]

        def compute(irefs, oref):
            def cg_body(cg, carry):
                row = 0
                for (l, stage, nf, nv), iref in zip(levels, irefs):
                    ivec = iref[pl.ds(cg * 16, 16)]
                    for f in range(nf):
                        v = plsc.load_gather(stage, [ivec + (f * nv)])
                        oref[row + f, pl.ds(cg * 16, 16)] = v
                    row += nf
                return carry

            lax.fori_loop(0, BBLK // 16, cg_body, 0)

        def odst_ref(oref):
            return oref.at[pl.ds(0, out_rows), :]

        def half(k, t, icur, inxt, ocur):
            wait_idxset(t, icur)
            pl.when(k > 0)(
                lambda: pltpu.make_async_copy(odst_ref(ocur), out_dst(t - 2),
                                              wsem).wait())
            compute(icur, ocur)
            pltpu.async_copy(odst_ref(ocur), out_dst(t), wsem)
            fire_idxset(t + 2, icur)

        ia = [(ib0.at[pl.ds(l * BBLK, BBLK)]) for l in range(nlv)]
        ib = [(ib1.at[pl.ds(l * BBLK, BBLK)]) for l in range(nlv)]
        fire_idxset(0, ia)
        fire_idxset(1, ib)

        def blk_body(k, carry):
            half(k, 2 * k, ia, ib, ob0)
            half(k, 2 * k + 1, ib, ia, ob1)
            return carry

        lax.fori_loop(0, slots // 2, blk_body, 0)
        wait_idxset(slots, ia)
        wait_idxset(slots + 1, ib)
        pltpu.make_async_copy(odst_ref(ob0), out_dst(slots - 2), wsem).wait()
        pltpu.make_async_copy(odst_ref(ob1), out_dst(slots - 1), wsem).wait()

    pl.when(wid < NA)(a_path)
    pl.when((wid >= NA) & (wid < NA + NB1))(
        lambda: b_path(wid - NA, NB1, SLOTS_B1,
                       ((0, st0, 16, 20), (1, st1, 32, 200),
                        (2, st2, 32, 2000)),
                       0, 80, 0))
    pl.when(wid >= NA + NB1)(
        lambda: b_path(wid - NA - NB1, NB2, SLOTS_B2,
                       ((2, st2, 32, 2000),),
                       80, 32, 1))


@jax.jit
def kernel(code_levels, W0, W1, W2, W3):
    mesh = plsc.VectorSubcoreMesh(core_axis_name="c", subcore_axis_name="s")
    f = pl.kernel(
        _body,
        out_type=jax.ShapeDtypeStruct((DTOT, N), jnp.float32),
        mesh=mesh,
        scratch_types=[
            pltpu.VMEM((ABLK,), jnp.int32),
            pltpu.VMEM((ABLK,), jnp.int32),
            pltpu.VMEM((3 * BBLK,), jnp.int32),
            pltpu.VMEM((3 * BBLK,), jnp.int32),
            pltpu.VMEM((ABLK, DIMS[3]), jnp.float32),
            pltpu.VMEM((ABLK, DIMS[3]), jnp.float32),
            pltpu.VMEM((128, ABLK), jnp.float32),
            pltpu.VMEM((128, ABLK), jnp.float32),
            pltpu.VMEM((80, BBLK), jnp.float32),
            pltpu.VMEM((80, BBLK), jnp.float32),
            pltpu.VMEM((DIMS[0] * 20,), jnp.float32),
            pltpu.VMEM((DIMS[1] * 200,), jnp.float32),
            pltpu.VMEM((32 * 2000,), jnp.float32),
            pltpu.SemaphoreType.DMA,
            pltpu.SemaphoreType.DMA,
            pltpu.SemaphoreType.DMA,
        ],
        compiler_params=pltpu.CompilerParams(
            use_tc_tiling_on_sc=False, needs_layout_passes=False),
    )
    outt = f(code_levels.T.reshape(-1), W0.T.reshape(-1), W1.T.reshape(-1),
             W2.T.reshape(-1), W3)
    return outt.T


# speedup vs baseline: 3.1113x; 2.6682x over previous
import functools

import jax
import jax.numpy as jnp
from jax import lax
from jax.experimental import pallas as pl
from jax.experimental.pallas import tpu as pltpu
from jax.experimental.pallas import tpu_sc as plsc

N = 50000
NLEV = 4
DIMS = (16, 32, 64, 128)
DTOT = 240
NC, NS = 2, 16
NW = NC * NS
NA, NB1, NB2 = 16, 11, 5

ABLK = 64
NBLK_A = -(-N // ABLK)
LAST_A = N - ABLK
SLOTS_A = 2 * (-(-NBLK_A // (2 * NA)))

BBLK = 128
NBLK_B = -(-N // BBLK)
LAST_B = N - BBLK
SLOTS_B1 = 2 * (-(-NBLK_B // (2 * NB1)))
SLOTS_B2 = 2 * (-(-NBLK_B // (2 * NB2)))


def _body(clvt, w0t, w1t, w2t, w3, outt,
          ia0, ia1, ib0, ib1, w3ra, w3rb, oa0, oa1, ob0, ob1,
          st0, st1, st2, isem, gsem, wsem):
    wid = lax.axis_index("c") * NS + lax.axis_index("s")
    iota = lax.iota(jnp.int32, 16)

    def dec(ref, n):
        for j in range(n // 16):
            ref[pl.ds(j * 16, 16)] = ref[pl.ds(j * 16, 16)] - 1


    def a_path():
        aid = wid

        def base_a(t):
            return lax.min(lax.min(t, SLOTS_A - 1) * NA + aid,
                           NBLK_A - 1) * ABLK

        def base_a_clamped(t):
            return lax.min(base_a(t), LAST_A)

        def idx_src(t):
            return clvt.at[pl.ds(3 * N + base_a_clamped(t), ABLK)]

        def fire_idx(t, iref):
            pltpu.async_copy(idx_src(t), iref, isem)

        def wait_idx(t, iref):
            pltpu.make_async_copy(idx_src(t), iref, isem).wait()

        def fire_gather(iref, wref):
            pltpu.async_copy(w3.at[iref], wref, gsem)

        def wait_gather(iref, wref):
            pltpu.make_async_copy(w3.at[iref], wref, gsem).wait()

        def out_dst(t):
            return outt.at[pl.ds(112, 128), pl.ds(base_a_clamped(t), ABLK)]

        def transpose(wref, oref):
            @functools.partial(plsc.parallel_loop, 0, ABLK // 16)
            def cg_body(cg):
                cvec = iota + cg * 16
                for f in range(128):
                    fvec = jnp.full((16,), f, jnp.int32)
                    v = plsc.load_gather(wref, [cvec, fvec])
                    oref[f, pl.ds(cg * 16, 16)] = v

        def half(k, t, icur, inxt, wcur, wnxt, ocur):
            wait_idx(t + 1, inxt)
            dec(inxt, ABLK)
            wait_gather(icur, wcur)
            fire_gather(inxt, wnxt)
            fire_idx(t + 2, icur)
            pl.when(k > 0)(
                lambda: pltpu.make_async_copy(ocur, out_dst(t - 2),
                                              wsem).wait())
            transpose(wcur, ocur)
            pltpu.async_copy(ocur, out_dst(t), wsem)

        pltpu.sync_copy(idx_src(0), ia0)
        dec(ia0, ABLK)
        fire_gather(ia0, w3ra)
        fire_idx(1, ia1)

        def blk_body(k, carry):
            half(k, 2 * k, ia0, ia1, w3ra, w3rb, oa0)
            half(k, 2 * k + 1, ia1, ia0, w3rb, w3ra, oa1)
            return carry

        lax.fori_loop(0, SLOTS_A // 2, blk_body, 0)
        pltpu.make_async_copy(idx_src(0), ia0, isem).wait()
        pltpu.make_async_copy(w3.at[ia0], w3ra, gsem).wait()
        pltpu.make_async_copy(oa0, out_dst(SLOTS_A - 2), wsem).wait()
        pltpu.make_async_copy(oa1, out_dst(SLOTS_A - 1), wsem).wait()


    def b_path(bid, n_g, slots, levels, out_off, out_rows, w2_half):
        pltpu.sync_copy(w0t, st0)
        pltpu.sync_copy(w1t, st1)
        pltpu.sync_copy(w2t.at[pl.ds(w2_half * 64000, 64000)], st2)
        nlv = len(levels)

        def base_b(t):
            return lax.min(
                lax.min(lax.min(t, slots - 1) * n_g + bid, NBLK_B - 1)
                * BBLK, LAST_B)

        def idx_src(t, l):
            return clvt.at[pl.ds(l * N + base_b(t), BBLK)]

        def fire_idxset(t, irefs):
            for (l, _, _, _), iref in zip(levels, irefs):
                pltpu.async_copy(idx_src(t, l), iref, isem)

        def wait_idxset(t, irefs):
            for (l, _, _, _), iref in zip(levels, irefs):
                pltpu.make_async_copy(idx_src(t, l), iref, isem).wait()
            for iref in irefs:
                dec(iref, BBLK)

        def out_dst(t):
            return outt.at[pl.ds(out_off, out_rows),
                           pl.ds(base_b(t), BBLK)]

        def compute(irefs, oref):
            @functools.partial(plsc.parallel_loop, 0, BBLK // 16)
            def cg_body(cg):
                row = 0
                for (l, stage, nf, nv), iref in zip(levels, irefs):
                    ivec = iref[pl.ds(cg * 16, 16)]
                    for f in range(nf):
                        v = plsc.load_gather(stage, [ivec + (f * nv)])
                        oref[row + f, pl.ds(cg * 16, 16)] = v
                    row += nf

        def odst_ref(oref):
            return oref.at[pl.ds(0, out_rows), :]

        def half(k, t, icur, inxt, ocur):
            wait_idxset(t, icur)
            pl.when(k > 0)(
                lambda: pltpu.make_async_copy(odst_ref(ocur), out_dst(t - 2),
                                              wsem).wait())
            compute(icur, ocur)
            pltpu.async_copy(odst_ref(ocur), out_dst(t), wsem)
            fire_idxset(t + 2, icur)

        ia = [(ib0.at[pl.ds(l * BBLK, BBLK)]) for l in range(nlv)]
        ib = [(ib1.at[pl.ds(l * BBLK, BBLK)]) for l in range(nlv)]
        fire_idxset(0, ia)
        fire_idxset(1, ib)

        def blk_body(k, carry):
            half(k, 2 * k, ia, ib, ob0)
            half(k, 2 * k + 1, ib, ia, ob1)
            return carry

        lax.fori_loop(0, slots // 2, blk_body, 0)
        wait_idxset(slots, ia)
        wait_idxset(slots + 1, ib)
        pltpu.make_async_copy(odst_ref(ob0), out_dst(slots - 2), wsem).wait()
        pltpu.make_async_copy(odst_ref(ob1), out_dst(slots - 1), wsem).wait()

    pl.when(wid < NA)(a_path)
    pl.when((wid >= NA) & (wid < NA + NB1))(
        lambda: b_path(wid - NA, NB1, SLOTS_B1,
                       ((0, st0, 16, 20), (1, st1, 32, 200),
                        (2, st2, 32, 2000)),
                       0, 80, 0))
    pl.when(wid >= NA + NB1)(
        lambda: b_path(wid - NA - NB1, NB2, SLOTS_B2,
                       ((2, st2, 32, 2000),),
                       80, 32, 1))


@jax.jit
def kernel(code_levels, W0, W1, W2, W3):
    mesh = plsc.VectorSubcoreMesh(core_axis_name="c", subcore_axis_name="s")
    f = pl.kernel(
        _body,
        out_type=jax.ShapeDtypeStruct((DTOT, N), jnp.float32),
        mesh=mesh,
        scratch_types=[
            pltpu.VMEM((ABLK,), jnp.int32),
            pltpu.VMEM((ABLK,), jnp.int32),
            pltpu.VMEM((3 * BBLK,), jnp.int32),
            pltpu.VMEM((3 * BBLK,), jnp.int32),
            pltpu.VMEM((ABLK, DIMS[3]), jnp.float32),
            pltpu.VMEM((ABLK, DIMS[3]), jnp.float32),
            pltpu.VMEM((128, ABLK), jnp.float32),
            pltpu.VMEM((128, ABLK), jnp.float32),
            pltpu.VMEM((80, BBLK), jnp.float32),
            pltpu.VMEM((80, BBLK), jnp.float32),
            pltpu.VMEM((DIMS[0] * 20,), jnp.float32),
            pltpu.VMEM((DIMS[1] * 200,), jnp.float32),
            pltpu.VMEM((32 * 2000,), jnp.float32),
            pltpu.SemaphoreType.DMA,
            pltpu.SemaphoreType.DMA,
            pltpu.SemaphoreType.DMA,
        ],
        compiler_params=pltpu.CompilerParams(
            use_tc_tiling_on_sc=False, needs_layout_passes=False),
    )
    outt = f(code_levels.T.reshape(-1), W0.T.reshape(-1), W1.T.reshape(-1),
             W2.T.reshape(-1), W3)
    return outt.T
